# baseline (speedup 1.0000x reference)
import jax
import jax.numpy as jnp
from jax import lax
from jax.experimental import pallas as pl
from jax.experimental.pallas import tpu as pltpu
from jax.experimental.pallas import tpu_sc as plsc

B = 16384
F = 16
NC = 2
NS = 16
L = 16
NW = NC * NS
BPW = B // NW
NWAVE = BPW // L
NB = 1000064


def _fm_body(user_hbm, item_hbm, uembT_hbm, iembT_hbm, biasT_hbm, w0_hbm,
             out_hbm,
             idx_u, idx_i, ublk, ublk2, iblk, bublk, biblk, w0_v, out_v,
             sem, sem_u2, sem_i, sem_b):
    wid = lax.axis_index("s") * NC + lax.axis_index("c")
    base = wid * BPW

    pltpu.sync_copy(user_hbm.at[pl.ds(base, BPW)], idx_u)
    pltpu.sync_copy(item_hbm.at[pl.ds(base, BPW)], idx_i)
    pltpu.sync_copy(w0_hbm, w0_v)

    w0vec = w0_v[...]
    mask127 = jnp.full((L,), 127, jnp.int32)

    def fire_u(w, buf, s):
        uvec = idx_u[pl.ds(w * L, L)]
        for j in range(L):
            ru = uvec[j]
            rbu = pl.multiple_of(
                lax.shift_left(lax.shift_right_logical(ru, 7), 7), 128)
            pltpu.async_copy(uembT_hbm.at[:, pl.ds(rbu, 128)], buf.at[j], s)

    def fire_i(w):
        uvec = idx_u[pl.ds(w * L, L)]
        ivec = idx_i[pl.ds(w * L, L)]
        for j in range(L):
            ru = uvec[j]
            ri = ivec[j]
            rbu = pl.multiple_of(
                lax.shift_left(lax.shift_right_logical(ru, 7), 7), 128)
            rbi = pl.multiple_of(
                lax.shift_left(lax.shift_right_logical(ri, 7), 7), 128)
            pltpu.async_copy(iembT_hbm.at[:, pl.ds(rbi, 128)], iblk.at[j], sem_i)
            pltpu.async_copy(biasT_hbm.at[:, pl.ds(rbu, 128)],
                             bublk.at[pl.ds(j, 1)], sem_b)
            pltpu.async_copy(biasT_hbm.at[:, pl.ds(rbi, 128)],
                             biblk.at[pl.ds(j, 1)], sem_b)

    def drain(buf, s):
        for j in range(L):
            pltpu.make_async_copy(
                uembT_hbm.at[:, pl.ds(0, 128)], buf.at[j], s).wait()

    def drain_b():
        for j in range(L):
            pltpu.make_async_copy(
                biasT_hbm.at[:, pl.ds(0, 128)], bublk.at[pl.ds(j, 1)], sem_b).wait()
            pltpu.make_async_copy(
                biasT_hbm.at[:, pl.ds(0, 128)], biblk.at[pl.ds(j, 1)], sem_b).wait()

    def compute(w, ub):
        uvec = idx_u[pl.ds(w * L, L)]
        ivec = idx_i[pl.ds(w * L, L)]
        lanes_u = uvec & mask127
        lanes_i = ivec & mask127
        el = lax.iota(jnp.int32, L)
        acc = jnp.zeros((L,), jnp.float32)
        for f in range(F):
            fv = jnp.full((L,), f, jnp.int32)
            acc = acc + (plsc.load_gather(ub, [el, fv, lanes_u])
                         * plsc.load_gather(iblk, [el, fv, lanes_i]))
        bu = plsc.load_gather(bublk, [el, lanes_u])
        bi = plsc.load_gather(biblk, [el, lanes_i])
        x = w0vec + bu + bi + acc
        out_v[pl.ds(w * L, L)] = 1.0 / (1.0 + jnp.exp(-x))

    fire_u(0, ublk, sem)

    def pair(hh, carry):
        w0 = 2 * hh
        w1 = w0 + 1
        fire_i(w0)
        fire_u(w1, ublk2, sem_u2)
        drain(ublk, sem)
        drain(iblk, sem_i)
        drain_b()
        compute(w0, ublk)
        fire_i(w1)

        @pl.when(hh < NWAVE // 2 - 1)
        def _():
            fire_u(w0 + 2, ublk, sem)

        drain(ublk2, sem_u2)
        drain(iblk, sem_i)
        drain_b()
        compute(w1, ublk2)
        return carry

    lax.fori_loop(0, NWAVE // 2, pair, 0)
    pltpu.sync_copy(out_v, out_hbm.at[pl.ds(base, BPW)])


@jax.jit
def kernel(user, item, user_emb, item_emb, bias_table, w0):
    user1d = user.astype(jnp.int32)
    item1d = item.astype(jnp.int32)
    uembT = user_emb.T
    iembT = item_emb.T
    biasT = bias_table.T
    w0v = jnp.broadcast_to(w0.astype(jnp.float32), (L,))

    fn = pl.kernel(
        _fm_body,
        out_type=jax.ShapeDtypeStruct((B,), jnp.float32),
        mesh=plsc.VectorSubcoreMesh(
            core_axis_name="c", subcore_axis_name="s",
            num_cores=NC, num_subcores=NS),
        scratch_types=[
            pltpu.VMEM((BPW,), jnp.int32),
            pltpu.VMEM((BPW,), jnp.int32),
            pltpu.VMEM((L, F, 128), jnp.float32),
            pltpu.VMEM((L, F, 128), jnp.float32),
            pltpu.VMEM((L, F, 128), jnp.float32),
            pltpu.VMEM((L, 128), jnp.float32),
            pltpu.VMEM((L, 128), jnp.float32),
            pltpu.VMEM((L,), jnp.float32),
            pltpu.VMEM((BPW,), jnp.float32),
            pltpu.SemaphoreType.DMA,
            pltpu.SemaphoreType.DMA,
            pltpu.SemaphoreType.DMA,
            pltpu.SemaphoreType.DMA,
        ],
        compiler_params=pltpu.CompilerParams(needs_layout_passes=False),
    )
    return fn(user1d, item1d, uembT, iembT, biasT, w0v)


# speedup vs baseline: 1.0578x; 1.0578x over previous
import jax
import jax.numpy as jnp
from jax import lax
from jax.experimental import pallas as pl
from jax.experimental.pallas import tpu as pltpu
from jax.experimental.pallas import tpu_sc as plsc

B = 16384
F = 16
NC = 2
NS = 16
L = 16
NW = NC * NS
BPW = B // NW
NWAVE = BPW // L
NB = 1000064


def _fm_body(user_hbm, item_hbm, uembT_hbm, iembT_hbm, biasT_hbm, w0_hbm,
             out_hbm,
             idx_u, idx_i, ublk, iblk, bvals_u, bvals_i, w0_v, out_v,
             sp_bias, sem, sem_b):
    wid = lax.axis_index("s") * NC + lax.axis_index("c")
    base = wid * BPW
    sid = lax.axis_index("s")

    pltpu.sync_copy(user_hbm.at[pl.ds(base, BPW)], idx_u)
    pltpu.sync_copy(item_hbm.at[pl.ds(base, BPW)], idx_i)
    pltpu.sync_copy(w0_hbm, w0_v)

    @pl.when(sid == 0)
    def _():
        pltpu.sync_copy(biasT_hbm.at[0, pl.ds(0, NB)], sp_bias)
    plsc.subcore_barrier()

    bias_copies = []
    for j in range(BPW // 128):
        sl = pl.ds(j * 128, 128)
        bias_copies.append(pltpu.async_copy(sp_bias.at[idx_u.at[sl]], bvals_u.at[sl], sem_b))
        bias_copies.append(pltpu.async_copy(sp_bias.at[idx_i.at[sl]], bvals_i.at[sl], sem_b))
    for c in bias_copies:
        c.wait()

    w0vec = w0_v[...]
    mask127 = jnp.full((L,), 127, jnp.int32)

    def wave(w, carry):
        uvec = idx_u[pl.ds(w * L, L)]
        ivec = idx_i[pl.ds(w * L, L)]
        copies = []
        for j in range(L):
            ru = uvec[j]
            ri = ivec[j]
            rbu = pl.multiple_of(
                lax.shift_left(lax.shift_right_logical(ru, 7), 7), 128)
            rbi = pl.multiple_of(
                lax.shift_left(lax.shift_right_logical(ri, 7), 7), 128)
            copies.append(pltpu.async_copy(
                uembT_hbm.at[:, pl.ds(rbu, 128)], ublk.at[j], sem))
            copies.append(pltpu.async_copy(
                iembT_hbm.at[:, pl.ds(rbi, 128)], iblk.at[j], sem))
        for c in copies:
            c.wait()

        lanes_u = uvec & mask127
        lanes_i = ivec & mask127
        el = lax.iota(jnp.int32, L)
        acc = jnp.zeros((L,), jnp.float32)
        for f in range(F):
            fv = jnp.full((L,), f, jnp.int32)
            acc = acc + (plsc.load_gather(ublk, [el, fv, lanes_u])
                         * plsc.load_gather(iblk, [el, fv, lanes_i]))
        grow = w * L + el
        bu = plsc.load_gather(bvals_u, [grow])
        bi = plsc.load_gather(bvals_i, [grow])
        x = w0vec + bu + bi + acc
        out_v[pl.ds(w * L, L)] = 1.0 / (1.0 + jnp.exp(-x))
        return carry

    lax.fori_loop(0, NWAVE, wave, 0)
    pltpu.sync_copy(out_v, out_hbm.at[pl.ds(base, BPW)])


@jax.jit
def kernel(user, item, user_emb, item_emb, bias_table, w0):
    user1d = user.astype(jnp.int32)
    item1d = item.astype(jnp.int32)
    uembT = user_emb.T
    iembT = item_emb.T
    biasT = bias_table.T
    w0v = jnp.broadcast_to(w0.astype(jnp.float32), (L,))

    fn = pl.kernel(
        _fm_body,
        out_type=jax.ShapeDtypeStruct((B,), jnp.float32),
        mesh=plsc.VectorSubcoreMesh(
            core_axis_name="c", subcore_axis_name="s",
            num_cores=NC, num_subcores=NS),
        scratch_types=[
            pltpu.VMEM((BPW,), jnp.int32),
            pltpu.VMEM((BPW,), jnp.int32),
            pltpu.VMEM((L, F, 128), jnp.float32),
            pltpu.VMEM((L, F, 128), jnp.float32),
            pltpu.VMEM((BPW,), jnp.float32),
            pltpu.VMEM((BPW,), jnp.float32),
            pltpu.VMEM((L,), jnp.float32),
            pltpu.VMEM((BPW,), jnp.float32),
            pltpu.VMEM_SHARED((NB,), jnp.float32),
            pltpu.SemaphoreType.DMA,
            pltpu.SemaphoreType.DMA,
        ],
        compiler_params=pltpu.CompilerParams(needs_layout_passes=False),
    )
    return fn(user1d, item1d, uembT, iembT, biasT, w0v)
